# baseline (speedup 1.0000x reference)
import dataclasses
import functools

import jax
import jax.numpy as jnp
from jax import lax
from jax.experimental import pallas as pl
from jax.experimental.pallas import tpu as pltpu
from jax.experimental.pallas import tpu_sc as plsc

NEG_SLOPE = 0.2
NC = 2
NS = 16
L = 16
K = 80


def _k1a(x, W, att_src, att_dst):
  n, d_in = x.shape
  d_out = W.shape[1]
  blk = 1000

  def body(x_ref, w_ref, asv_ref, adv_ref, h_ref, as_ref, ad_ref):
    hb = jnp.dot(x_ref[...], w_ref[...], preferred_element_type=jnp.float32)
    h_ref[...] = hb
    as_ref[...] = jnp.sum(hb * asv_ref[...], axis=1, keepdims=True)
    ad_ref[...] = jnp.sum(hb * adv_ref[...], axis=1, keepdims=True)

  return pl.pallas_call(
      body,
      grid=(n // blk,),
      in_specs=[
          pl.BlockSpec((blk, d_in), lambda i: (i, 0)),
          pl.BlockSpec((d_in, d_out), lambda i: (0, 0)),
          pl.BlockSpec((1, d_out), lambda i: (0, 0)),
          pl.BlockSpec((1, d_out), lambda i: (0, 0)),
      ],
      out_specs=[
          pl.BlockSpec((blk, d_out), lambda i: (i, 0)),
          pl.BlockSpec((blk, 1), lambda i: (i, 0)),
          pl.BlockSpec((blk, 1), lambda i: (i, 0)),
      ],
      out_shape=[
          jax.ShapeDtypeStruct((n, d_out), jnp.float32),
          jax.ShapeDtypeStruct((n, 1), jnp.float32),
          jax.ShapeDtypeStruct((n, 1), jnp.float32),
      ],
  )(x, W, att_src.reshape(1, -1), att_dst.reshape(1, -1))


def _k1b(edge_attr, W_edge, att_edge):
  e, d_edge = edge_attr.shape
  blk = 6400

  def body(ea_ref, we_ref, aev_ref, ae_ref, sum_ref):
    w_row = lax.dot_general(aev_ref[...], we_ref[...],
                            (((1,), (1,)), ((), ())),
                            preferred_element_type=jnp.float32)
    out = lax.dot_general(w_row, ea_ref[...],
                          (((1,), (1,)), ((), ())),
                          preferred_element_type=jnp.float32)
    off = pl.multiple_of(pl.program_id(0) * blk, 128)
    ae_ref[:, pl.ds(off, blk)] = out

    @pl.when(pl.program_id(0) == 0)
    def _():
      sum_ref[...] = jnp.zeros((1, 1), jnp.float32)

    sum_ref[...] += jnp.sum(out, keepdims=True)

  return pl.pallas_call(
      body,
      grid=(e // blk,),
      in_specs=[
          pl.BlockSpec((blk, d_edge), lambda i: (i, 0)),
          pl.BlockSpec(W_edge.shape, lambda i: (0, 0)),
          pl.BlockSpec((1, att_edge.shape[0]), lambda i: (0, 0)),
      ],
      out_specs=[
          pl.BlockSpec((1, e), lambda i: (0, 0)),
          pl.BlockSpec((1, 1), lambda i: (0, 0)),
      ],
      out_shape=[
          jax.ShapeDtypeStruct((1, e), jnp.float32),
          jax.ShapeDtypeStruct((1, 1), jnp.float32),
      ],
  )(edge_attr, W_edge, att_edge.reshape(1, -1))


def _sc_edges(h, src, dst, a_edge, a_src, a_dst, n_pad):
  n, d = h.shape
  e = src.shape[0]
  nw = NC * NS
  ew = e // nw
  nb = ew // K
  assert e == nw * ew and ew == nb * K and nb % 2 == 1
  rows_per_tile = n_pad // NS
  assert rows_per_tile % K == 0 and rows_per_tile % 8 == 0

  mesh = plsc.VectorSubcoreMesh(core_axis_name="c", subcore_axis_name="s")
  cp = pltpu.CompilerParams()
  if "needs_layout_passes" in pltpu.CompilerParams.__dataclass_fields__:
    cp = dataclasses.replace(cp, needs_layout_passes=False)

  @functools.partial(
      pl.kernel,
      out_type=[
          jax.ShapeDtypeStruct((NC, n_pad, d), jnp.float32),
          jax.ShapeDtypeStruct((NC, n_pad), jnp.float32),
      ],
      mesh=mesh,
      compiler_params=cp,
      scratch_types=(
          [pltpu.VMEM((K,), jnp.int32)] * 4 +
          [pltpu.VMEM((K,), jnp.int32)] * 2 +
          [pltpu.VMEM((K,), jnp.float32)] * 6 +
          [pltpu.VMEM((K,), jnp.float32)] * 2 +
          [pltpu.VMEM((K, d), jnp.float32)] * 2 +
          [pltpu.VMEM((rows_per_tile,), jnp.float32)] +
          [pltpu.VMEM_SHARED((n_pad, d), jnp.float32),
           pltpu.VMEM_SHARED((n_pad,), jnp.float32)] +
          [pltpu.SemaphoreType.DMA] * 6
      ),
  )
  def sc_kernel(h_hbm, src_hbm, dst_hbm, ae_hbm, asrc_hbm, adst_hbm,
                out_hbm, den_hbm,
                src0, src1, dst0, dst1, sd0, sd1,
                ae0, ae1, as0, as1, ad0, ad1, s0, s1,
                rows0, rows1, zden_v, acc_sh, den_sh,
                semi0, semi1, semg0, semg1, semsc0, semsc1):
    cid = lax.axis_index("c")
    sid = lax.axis_index("s")
    wid = cid * NS + sid

    srcs, dsts, sds = (src0, src1), (dst0, dst1), (sd0, sd1)
    aes, asvs, advs, svs = (ae0, ae1), (as0, as1), (ad0, ad1), (s0, s1)
    rowss = (rows0, rows1)
    semi, semg, semsc = (semi0, semi1), (semg0, semg1), (semsc0, semsc1)

    zeros_v = jnp.zeros((L,), jnp.float32)
    @pl.loop(0, K)
    def _(r):
      for c in range(d // L):
        rows0.at[r][pl.ds(c * L, L)] = zeros_v

    @pl.loop(0, rows_per_tile, step=L)
    def _(i):
      zden_v[pl.ds(i, L)] = zeros_v

    @pl.loop(0, rows_per_tile // K)
    def _(u):
      pltpu.sync_copy(
          rows0, acc_sh.at[pl.ds(sid * rows_per_tile + u * K, K)])

    pltpu.sync_copy(
        zden_v, den_sh.at[pl.ds(sid * rows_per_tile, rows_per_tile)])

    plsc.subcore_barrier()

    base_w = wid * ew

    def start_idx(b, p):
      base = base_w + b * K
      pltpu.async_copy(src_hbm.at[pl.ds(base, K)], srcs[p], semi[p])
      pltpu.async_copy(dst_hbm.at[pl.ds(base, K)], dsts[p], semi[p])

    def wait_idx(p):
      pltpu.make_async_copy(src_hbm.at[pl.ds(0, K)], srcs[p], semi[p]).wait()
      pltpu.make_async_copy(dst_hbm.at[pl.ds(0, K)], dsts[p], semi[p]).wait()

    def start_gathers(b, p):
      base = base_w + b * K
      pltpu.async_copy(ae_hbm.at[pl.ds(base, K)], aes[p], semg[p])
      pltpu.async_copy(dst_hbm.at[pl.ds(base, K)], sds[p], semg[p])
      pltpu.async_copy(asrc_hbm.at[srcs[p]], asvs[p], semg[p])
      pltpu.async_copy(adst_hbm.at[dsts[p]], advs[p], semg[p])
      pltpu.async_copy(h_hbm.at[srcs[p]], rowss[p], semg[p])

    def wait_gathers(p):
      pltpu.make_async_copy(ae_hbm.at[pl.ds(0, K)], aes[p], semg[p]).wait()
      pltpu.make_async_copy(dst_hbm.at[pl.ds(0, K)], sds[p], semg[p]).wait()
      pltpu.make_async_copy(asrc_hbm.at[srcs[p]], asvs[p], semg[p]).wait()
      pltpu.make_async_copy(adst_hbm.at[dsts[p]], advs[p], semg[p]).wait()
      pltpu.make_async_copy(h_hbm.at[srcs[p]], rowss[p], semg[p]).wait()

    def compute(p):
      for j in range(0, K, L):
        al = asvs[p][pl.ds(j, L)] + advs[p][pl.ds(j, L)] + aes[p][pl.ds(j, L)]
        al = jnp.maximum(al, al * NEG_SLOPE)
        svs[p][pl.ds(j, L)] = jnp.exp(al)

      @pl.loop(0, K, step=4)
      def _(r0):
        for u in range(4):
          r = r0 + u
          ridx = jnp.zeros((L,), jnp.int32) + r
          ssplat = plsc.load_gather(svs[p], [ridx])
          row = rowss[p].at[r]
          for c in range(d // L):
            sl = pl.ds(c * L, L)
            row[sl] = row[sl] * ssplat

    def start_scatter(p):
      pltpu.async_copy(rowss[p], acc_sh.at[sds[p]], semsc[p], add=True)
      pltpu.async_copy(svs[p], den_sh.at[sds[p]], semsc[p], add=True)

    def wait_scatter(p):
      pltpu.make_async_copy(rowss[p], acc_sh.at[sds[p]], semsc[p]).wait()
      pltpu.make_async_copy(svs[p], den_sh.at[sds[p]], semsc[p]).wait()

    start_idx(0, 0)
    start_idx(1, 1)
    wait_idx(0)
    start_gathers(0, 0)

    @pl.loop(0, (nb - 1) // 2)
    def _(g):
      for p in (0, 1):
        b = 2 * g + p
        q = 1 - p
        wait_gathers(p)

        @pl.when(b + 2 < nb)
        def _():
          start_idx(b + 2, p)

        compute(p)
        start_scatter(p)
        wait_idx(q)

        @pl.when(b >= 1)
        def _():
          wait_scatter(q)

        start_gathers(b + 1, q)

    wait_gathers(0)
    compute(0)
    start_scatter(0)
    wait_scatter(1)
    wait_scatter(0)

    plsc.subcore_barrier()

    stripe = pl.ds(sid * rows_per_tile, rows_per_tile)
    pltpu.sync_copy(acc_sh.at[stripe], out_hbm.at[cid].at[stripe])

    @pl.when(sid == 0)
    def _():
      pltpu.sync_copy(den_sh, den_hbm.at[cid])

  return sc_kernel(h, src, dst, a_edge, a_src, a_dst)


def _k2(out_p, h, a_src, a_dst, den_t, ae_sum, e_total):
  n, d = h.shape
  blk = 1000

  def body(p0_ref, p1_ref, h_ref, as_ref, ad_ref, dt_ref, sum_ref, o_ref):
    ae_mean = sum_ref[...] * (1.0 / e_total)
    v = as_ref[...] + ad_ref[...] + ae_mean
    v = jnp.maximum(v, v * NEG_SLOPE)
    s_self = jnp.exp(v)
    den = jnp.sum(dt_ref[...], axis=1, keepdims=True) + s_self
    numer = p0_ref[0] + p1_ref[0] + s_self * h_ref[...]
    o_ref[...] = numer / (den + 1e-16)

  return pl.pallas_call(
      body,
      grid=(n // blk,),
      in_specs=[
          pl.BlockSpec((1, blk, d), lambda i: (0, i, 0)),
          pl.BlockSpec((1, blk, d), lambda i: (1, i, 0)),
          pl.BlockSpec((blk, d), lambda i: (i, 0)),
          pl.BlockSpec((blk, 1), lambda i: (i, 0)),
          pl.BlockSpec((blk, 1), lambda i: (i, 0)),
          pl.BlockSpec((blk, NC), lambda i: (i, 0)),
          pl.BlockSpec((1, 1), lambda i: (0, 0)),
      ],
      out_specs=pl.BlockSpec((blk, d), lambda i: (i, 0)),
      out_shape=jax.ShapeDtypeStruct((n, d), jnp.float32),
  )(out_p, out_p, h, a_src, a_dst, den_t, ae_sum)


def kernel(x, edge_index, edge_attr, W, att_src, att_dst, W_edge, att_edge):
  e = edge_attr.shape[0]
  h, a_src2, a_dst2 = _k1a(x, W, att_src, att_dst)
  ae_row, ae_sum = _k1b(edge_attr, W_edge, att_edge)
  src = edge_index[0]
  dst = edge_index[1]
  n = h.shape[0]
  n_pad = 10240 if n == 10000 else ((n + 8 * NS - 1) // (8 * NS)) * 8 * NS
  out_p, den_p = _sc_edges(h, src, dst, ae_row.reshape(-1),
                           a_src2.reshape(-1), a_dst2.reshape(-1), n_pad)
  return _k2(out_p, h, a_src2, a_dst2, den_p.T, ae_sum, e)


# speedup vs baseline: 1.0231x; 1.0231x over previous
import dataclasses
import functools

import jax
import jax.numpy as jnp
from jax import lax
from jax.experimental import pallas as pl
from jax.experimental.pallas import tpu as pltpu
from jax.experimental.pallas import tpu_sc as plsc

NEG_SLOPE = 0.2
NC = 2
NS = 16
L = 16
K = 80


def _k1a(x, W, att_src, att_dst):
  n, d_in = x.shape
  d_out = W.shape[1]
  blk = 1000

  def body(x_ref, w_ref, asv_ref, adv_ref, h_ref, as_ref, ad_ref):
    hb = jnp.dot(x_ref[...], w_ref[...], preferred_element_type=jnp.float32)
    h_ref[...] = hb
    as_ref[...] = jnp.sum(hb * asv_ref[...], axis=1, keepdims=True)
    ad_ref[...] = jnp.sum(hb * adv_ref[...], axis=1, keepdims=True)

  return pl.pallas_call(
      body,
      grid=(n // blk,),
      in_specs=[
          pl.BlockSpec((blk, d_in), lambda i: (i, 0)),
          pl.BlockSpec((d_in, d_out), lambda i: (0, 0)),
          pl.BlockSpec((1, d_out), lambda i: (0, 0)),
          pl.BlockSpec((1, d_out), lambda i: (0, 0)),
      ],
      out_specs=[
          pl.BlockSpec((blk, d_out), lambda i: (i, 0)),
          pl.BlockSpec((blk, 1), lambda i: (i, 0)),
          pl.BlockSpec((blk, 1), lambda i: (i, 0)),
      ],
      out_shape=[
          jax.ShapeDtypeStruct((n, d_out), jnp.float32),
          jax.ShapeDtypeStruct((n, 1), jnp.float32),
          jax.ShapeDtypeStruct((n, 1), jnp.float32),
      ],
  )(x, W, att_src.reshape(1, -1), att_dst.reshape(1, -1))


def _k1b(edge_attr, W_edge, att_edge):
  e, d_edge = edge_attr.shape
  blk = 6400

  def body(ea_ref, we_ref, aev_ref, ae_ref, sum_ref):
    w_row = lax.dot_general(aev_ref[...], we_ref[...],
                            (((1,), (1,)), ((), ())),
                            preferred_element_type=jnp.float32)
    out = lax.dot_general(w_row, ea_ref[...],
                          (((1,), (1,)), ((), ())),
                          preferred_element_type=jnp.float32)
    off = pl.multiple_of(pl.program_id(0) * blk, 128)
    ae_ref[:, pl.ds(off, blk)] = out

    @pl.when(pl.program_id(0) == 0)
    def _():
      sum_ref[...] = jnp.zeros((1, 1), jnp.float32)

    sum_ref[...] += jnp.sum(out, keepdims=True)

  return pl.pallas_call(
      body,
      grid=(e // blk,),
      in_specs=[
          pl.BlockSpec((blk, d_edge), lambda i: (i, 0)),
          pl.BlockSpec(W_edge.shape, lambda i: (0, 0)),
          pl.BlockSpec((1, att_edge.shape[0]), lambda i: (0, 0)),
      ],
      out_specs=[
          pl.BlockSpec((1, e), lambda i: (0, 0)),
          pl.BlockSpec((1, 1), lambda i: (0, 0)),
      ],
      out_shape=[
          jax.ShapeDtypeStruct((1, e), jnp.float32),
          jax.ShapeDtypeStruct((1, 1), jnp.float32),
      ],
  )(edge_attr, W_edge, att_edge.reshape(1, -1))


def _sc_edges(h, src, dst, a_edge, a_src, a_dst, n_pad):
  n, d = h.shape
  e = src.shape[0]
  nw = NC * NS
  ew = e // nw
  nb = ew // K
  assert e == nw * ew and ew == nb * K and nb % 2 == 1
  rows_per_tile = n_pad // NS
  assert rows_per_tile % K == 0 and rows_per_tile % 8 == 0

  mesh = plsc.VectorSubcoreMesh(core_axis_name="c", subcore_axis_name="s")
  cp = pltpu.CompilerParams()
  if "needs_layout_passes" in pltpu.CompilerParams.__dataclass_fields__:
    cp = dataclasses.replace(cp, needs_layout_passes=False)

  @functools.partial(
      pl.kernel,
      out_type=[
          jax.ShapeDtypeStruct((NC, n_pad, d), jnp.float32),
          jax.ShapeDtypeStruct((NC, n_pad), jnp.float32),
      ],
      mesh=mesh,
      compiler_params=cp,
      scratch_types=(
          [pltpu.VMEM((K,), jnp.int32)] * 4 +
          [pltpu.VMEM((K,), jnp.int32)] * 2 +
          [pltpu.VMEM((K,), jnp.float32)] * 2 +
          [pltpu.VMEM((K,), jnp.float32)] * 2 +
          [pltpu.VMEM((n,), jnp.float32)] * 2 +
          [pltpu.VMEM((K, d), jnp.float32)] * 2 +
          [pltpu.VMEM((rows_per_tile,), jnp.float32)] +
          [pltpu.VMEM_SHARED((n_pad, d), jnp.float32),
           pltpu.VMEM_SHARED((n_pad,), jnp.float32)] +
          [pltpu.SemaphoreType.DMA] * 6
      ),
  )
  def sc_kernel(h_hbm, src_hbm, dst_hbm, ae_hbm, asrc_hbm, adst_hbm,
                out_hbm, den_hbm,
                src0, src1, dst0, dst1, sd0, sd1,
                ae0, ae1, s0, s1, asrc_v, adst_v,
                rows0, rows1, zden_v, acc_sh, den_sh,
                semi0, semi1, semg0, semg1, semsc0, semsc1):
    cid = lax.axis_index("c")
    sid = lax.axis_index("s")
    wid = cid * NS + sid

    srcs, dsts, sds = (src0, src1), (dst0, dst1), (sd0, sd1)
    aes, svs = (ae0, ae1), (s0, s1)
    rowss = (rows0, rows1)
    semi, semg, semsc = (semi0, semi1), (semg0, semg1), (semsc0, semsc1)

    zeros_v = jnp.zeros((L,), jnp.float32)
    @pl.loop(0, K)
    def _(r):
      for c in range(d // L):
        rows0.at[r][pl.ds(c * L, L)] = zeros_v

    @pl.loop(0, rows_per_tile, step=L)
    def _(i):
      zden_v[pl.ds(i, L)] = zeros_v

    @pl.loop(0, rows_per_tile // K)
    def _(u):
      pltpu.sync_copy(
          rows0, acc_sh.at[pl.ds(sid * rows_per_tile + u * K, K)])

    pltpu.sync_copy(
        zden_v, den_sh.at[pl.ds(sid * rows_per_tile, rows_per_tile)])

    pltpu.sync_copy(asrc_hbm, asrc_v)
    pltpu.sync_copy(adst_hbm, adst_v)

    plsc.subcore_barrier()

    base_w = wid * ew

    def start_idx(b, p):
      base = base_w + b * K
      pltpu.async_copy(src_hbm.at[pl.ds(base, K)], srcs[p], semi[p])
      pltpu.async_copy(dst_hbm.at[pl.ds(base, K)], dsts[p], semi[p])

    def wait_idx(p):
      pltpu.make_async_copy(src_hbm.at[pl.ds(0, K)], srcs[p], semi[p]).wait()
      pltpu.make_async_copy(dst_hbm.at[pl.ds(0, K)], dsts[p], semi[p]).wait()

    def start_gathers(b, p):
      base = base_w + b * K
      pltpu.async_copy(ae_hbm.at[pl.ds(base, K)], aes[p], semg[p])
      pltpu.async_copy(dst_hbm.at[pl.ds(base, K)], sds[p], semg[p])
      pltpu.async_copy(h_hbm.at[srcs[p]], rowss[p], semg[p])

    def wait_gathers(p):
      pltpu.make_async_copy(ae_hbm.at[pl.ds(0, K)], aes[p], semg[p]).wait()
      pltpu.make_async_copy(dst_hbm.at[pl.ds(0, K)], sds[p], semg[p]).wait()
      pltpu.make_async_copy(h_hbm.at[srcs[p]], rowss[p], semg[p]).wait()

    def compute(p):
      for j in range(0, K, L):
        si = srcs[p][pl.ds(j, L)]
        di = dsts[p][pl.ds(j, L)]
        a_s = plsc.load_gather(asrc_v, [si])
        a_d = plsc.load_gather(adst_v, [di])
        al = a_s + a_d + aes[p][pl.ds(j, L)]
        al = jnp.maximum(al, al * NEG_SLOPE)
        svs[p][pl.ds(j, L)] = jnp.exp(al)

      @pl.loop(0, K, step=4)
      def _(r0):
        for u in range(4):
          r = r0 + u
          ridx = jnp.zeros((L,), jnp.int32) + r
          ssplat = plsc.load_gather(svs[p], [ridx])
          row = rowss[p].at[r]
          for c in range(d // L):
            sl = pl.ds(c * L, L)
            row[sl] = row[sl] * ssplat

    def start_scatter(p):
      pltpu.async_copy(rowss[p], acc_sh.at[sds[p]], semsc[p], add=True)
      pltpu.async_copy(svs[p], den_sh.at[sds[p]], semsc[p], add=True)

    def wait_scatter(p):
      pltpu.make_async_copy(rowss[p], acc_sh.at[sds[p]], semsc[p]).wait()
      pltpu.make_async_copy(svs[p], den_sh.at[sds[p]], semsc[p]).wait()

    start_idx(0, 0)
    start_idx(1, 1)
    wait_idx(0)
    start_gathers(0, 0)

    @pl.loop(0, (nb - 1) // 2)
    def _(g):
      for p in (0, 1):
        b = 2 * g + p
        q = 1 - p
        wait_gathers(p)
        compute(p)

        @pl.when(b + 2 < nb)
        def _():
          start_idx(b + 2, p)

        start_scatter(p)
        wait_idx(q)

        @pl.when(b >= 1)
        def _():
          wait_scatter(q)

        start_gathers(b + 1, q)

    wait_gathers(0)
    compute(0)
    start_scatter(0)
    wait_scatter(1)
    wait_scatter(0)

    plsc.subcore_barrier()

    stripe = pl.ds(sid * rows_per_tile, rows_per_tile)
    pltpu.sync_copy(acc_sh.at[stripe], out_hbm.at[cid].at[stripe])

    @pl.when(sid == 0)
    def _():
      pltpu.sync_copy(den_sh, den_hbm.at[cid])

  return sc_kernel(h, src, dst, a_edge, a_src, a_dst)


def _k2(out_p, h, a_src, a_dst, den_t, ae_sum, e_total):
  n, d = h.shape
  blk = 1000

  def body(p0_ref, p1_ref, h_ref, as_ref, ad_ref, dt_ref, sum_ref, o_ref):
    ae_mean = sum_ref[...] * (1.0 / e_total)
    v = as_ref[...] + ad_ref[...] + ae_mean
    v = jnp.maximum(v, v * NEG_SLOPE)
    s_self = jnp.exp(v)
    den = jnp.sum(dt_ref[...], axis=1, keepdims=True) + s_self
    numer = p0_ref[0] + p1_ref[0] + s_self * h_ref[...]
    o_ref[...] = numer / (den + 1e-16)

  return pl.pallas_call(
      body,
      grid=(n // blk,),
      in_specs=[
          pl.BlockSpec((1, blk, d), lambda i: (0, i, 0)),
          pl.BlockSpec((1, blk, d), lambda i: (1, i, 0)),
          pl.BlockSpec((blk, d), lambda i: (i, 0)),
          pl.BlockSpec((blk, 1), lambda i: (i, 0)),
          pl.BlockSpec((blk, 1), lambda i: (i, 0)),
          pl.BlockSpec((blk, NC), lambda i: (i, 0)),
          pl.BlockSpec((1, 1), lambda i: (0, 0)),
      ],
      out_specs=pl.BlockSpec((blk, d), lambda i: (i, 0)),
      out_shape=jax.ShapeDtypeStruct((n, d), jnp.float32),
  )(out_p, out_p, h, a_src, a_dst, den_t, ae_sum)


def kernel(x, edge_index, edge_attr, W, att_src, att_dst, W_edge, att_edge):
  e = edge_attr.shape[0]
  h, a_src2, a_dst2 = _k1a(x, W, att_src, att_dst)
  ae_row, ae_sum = _k1b(edge_attr, W_edge, att_edge)
  src = edge_index[0]
  dst = edge_index[1]
  n = h.shape[0]
  n_pad = 10240 if n == 10000 else ((n + 8 * NS - 1) // (8 * NS)) * 8 * NS
  out_p, den_p = _sc_edges(h, src, dst, ae_row.reshape(-1),
                           a_src2.reshape(-1), a_dst2.reshape(-1), n_pad)
  return _k2(out_p, h, a_src2, a_dst2, den_p.T, ae_sum, e)


# speedup vs baseline: 1.2046x; 1.1774x over previous
import dataclasses
import functools

import jax
import jax.numpy as jnp
from jax import lax
from jax.experimental import pallas as pl
from jax.experimental.pallas import tpu as pltpu
from jax.experimental.pallas import tpu_sc as plsc

NEG_SLOPE = 0.2
NC = 2
NS = 16
L = 16
K = 80


def _k1a(x, W, att_src, att_dst):
  n, d_in = x.shape
  d_out = W.shape[1]
  blk = 1000

  def body(x_ref, w_ref, asv_ref, adv_ref, h_ref, as_ref, ad_ref):
    hb = jnp.dot(x_ref[...], w_ref[...], preferred_element_type=jnp.float32)
    h_ref[...] = hb
    as_ref[...] = jnp.sum(hb * asv_ref[...], axis=1, keepdims=True)
    ad_ref[...] = jnp.sum(hb * adv_ref[...], axis=1, keepdims=True)

  return pl.pallas_call(
      body,
      grid=(n // blk,),
      in_specs=[
          pl.BlockSpec((blk, d_in), lambda i: (i, 0)),
          pl.BlockSpec((d_in, d_out), lambda i: (0, 0)),
          pl.BlockSpec((1, d_out), lambda i: (0, 0)),
          pl.BlockSpec((1, d_out), lambda i: (0, 0)),
      ],
      out_specs=[
          pl.BlockSpec((blk, d_out), lambda i: (i, 0)),
          pl.BlockSpec((blk, 1), lambda i: (i, 0)),
          pl.BlockSpec((blk, 1), lambda i: (i, 0)),
      ],
      out_shape=[
          jax.ShapeDtypeStruct((n, d_out), jnp.float32),
          jax.ShapeDtypeStruct((n, 1), jnp.float32),
          jax.ShapeDtypeStruct((n, 1), jnp.float32),
      ],
  )(x, W, att_src.reshape(1, -1), att_dst.reshape(1, -1))


def _k1b(edge_attr, W_edge, att_edge):
  e, d_edge = edge_attr.shape
  blk = 6400

  def body(ea_ref, we_ref, aev_ref, ae_ref, sum_ref):
    w_row = lax.dot_general(aev_ref[...], we_ref[...],
                            (((1,), (1,)), ((), ())),
                            preferred_element_type=jnp.float32)
    out = lax.dot_general(w_row, ea_ref[...],
                          (((1,), (1,)), ((), ())),
                          preferred_element_type=jnp.float32)
    off = pl.multiple_of(pl.program_id(0) * blk, 128)
    ae_ref[:, pl.ds(off, blk)] = out

    @pl.when(pl.program_id(0) == 0)
    def _():
      sum_ref[...] = jnp.zeros((1, 1), jnp.float32)

    sum_ref[...] += jnp.sum(out, keepdims=True)

  return pl.pallas_call(
      body,
      grid=(e // blk,),
      in_specs=[
          pl.BlockSpec((blk, d_edge), lambda i: (i, 0)),
          pl.BlockSpec(W_edge.shape, lambda i: (0, 0)),
          pl.BlockSpec((1, att_edge.shape[0]), lambda i: (0, 0)),
      ],
      out_specs=[
          pl.BlockSpec((1, e), lambda i: (0, 0)),
          pl.BlockSpec((1, 1), lambda i: (0, 0)),
      ],
      out_shape=[
          jax.ShapeDtypeStruct((1, e), jnp.float32),
          jax.ShapeDtypeStruct((1, 1), jnp.float32),
      ],
  )(edge_attr, W_edge, att_edge.reshape(1, -1))


def _sc_edges(h, src, dst, a_edge, a_src, a_dst, n_pad):
  n, d = h.shape
  e = src.shape[0]
  nw = NC * NS
  ew = e // nw
  nb = ew // K
  assert e == nw * ew and ew == nb * K and nb % 2 == 1
  rows_per_tile = n_pad // NS
  assert rows_per_tile % K == 0 and rows_per_tile % 8 == 0

  mesh = plsc.VectorSubcoreMesh(core_axis_name="c", subcore_axis_name="s")
  cp = pltpu.CompilerParams()
  if "needs_layout_passes" in pltpu.CompilerParams.__dataclass_fields__:
    cp = dataclasses.replace(cp, needs_layout_passes=False)

  @functools.partial(
      pl.kernel,
      out_type=[
          jax.ShapeDtypeStruct((NC, n_pad, d), jnp.float32),
          jax.ShapeDtypeStruct((NC, n_pad), jnp.float32),
      ],
      mesh=mesh,
      compiler_params=cp,
      scratch_types=(
          [pltpu.VMEM((K,), jnp.int32)] * 4 +
          [pltpu.VMEM((K,), jnp.int32)] * 2 +
          [pltpu.VMEM((K,), jnp.float32)] * 2 +
          [pltpu.VMEM((K,), jnp.float32)] * 2 +
          [pltpu.VMEM((n,), jnp.float32)] * 2 +
          [pltpu.VMEM((K, d), jnp.float32)] * 2 +
          [pltpu.VMEM((rows_per_tile,), jnp.float32)] +
          [pltpu.VMEM_SHARED((n_pad, d), jnp.float32),
           pltpu.VMEM_SHARED((n_pad,), jnp.float32)] +
          [pltpu.SemaphoreType.DMA] * 6
      ),
  )
  def sc_kernel(h_hbm, src_hbm, dst_hbm, ae_hbm, asrc_hbm, adst_hbm,
                out_hbm, den_hbm,
                src0, src1, dst0, dst1, sd0, sd1,
                ae0, ae1, s0, s1, asrc_v, adst_v,
                rows0, rows1, zden_v, acc_sh, den_sh,
                semi0, semi1, semg0, semg1, semsc0, semsc1):
    cid = lax.axis_index("c")
    sid = lax.axis_index("s")
    wid = cid * NS + sid

    srcs, dsts, sds = (src0, src1), (dst0, dst1), (sd0, sd1)
    aes, svs = (ae0, ae1), (s0, s1)
    rowss = (rows0, rows1)
    semi, semg, semsc = (semi0, semi1), (semg0, semg1), (semsc0, semsc1)

    zeros_v = jnp.zeros((L,), jnp.float32)
    @pl.loop(0, K)
    def _(r):
      for c in range(d // L):
        rows0.at[r][pl.ds(c * L, L)] = zeros_v

    @pl.loop(0, rows_per_tile, step=L)
    def _(i):
      zden_v[pl.ds(i, L)] = zeros_v

    @pl.loop(0, rows_per_tile // K)
    def _(u):
      pltpu.sync_copy(
          rows0, acc_sh.at[pl.ds(sid * rows_per_tile + u * K, K)])

    pltpu.sync_copy(
        zden_v, den_sh.at[pl.ds(sid * rows_per_tile, rows_per_tile)])

    pltpu.sync_copy(asrc_hbm, asrc_v)
    pltpu.sync_copy(adst_hbm, adst_v)

    plsc.subcore_barrier()

    base_w = wid * ew

    def start_idx(b, p):
      base = base_w + b * K
      pltpu.async_copy(src_hbm.at[pl.ds(base, K)], srcs[p], semi[p])
      pltpu.async_copy(dst_hbm.at[pl.ds(base, K)], dsts[p], semi[p])

    def wait_idx(p):
      pltpu.make_async_copy(src_hbm.at[pl.ds(0, K)], srcs[p], semi[p]).wait()
      pltpu.make_async_copy(dst_hbm.at[pl.ds(0, K)], dsts[p], semi[p]).wait()

    def start_gathers(b, p):
      base = base_w + b * K
      pltpu.async_copy(ae_hbm.at[pl.ds(base, K)], aes[p], semg[p])
      pltpu.async_copy(dst_hbm.at[pl.ds(base, K)], sds[p], semg[p])
      pltpu.async_copy(h_hbm.at[srcs[p]], rowss[p], semg[p])

    def wait_gathers(p):
      pltpu.make_async_copy(ae_hbm.at[pl.ds(0, K)], aes[p], semg[p]).wait()
      pltpu.make_async_copy(dst_hbm.at[pl.ds(0, K)], sds[p], semg[p]).wait()
      pltpu.make_async_copy(h_hbm.at[srcs[p]], rowss[p], semg[p]).wait()

    def compute(p):
      for j in range(0, K, L):
        si = srcs[p][pl.ds(j, L)]
        di = dsts[p][pl.ds(j, L)]
        a_s = plsc.load_gather(asrc_v, [si])
        a_d = plsc.load_gather(adst_v, [di])
        al = a_s + a_d + aes[p][pl.ds(j, L)]
        al = jnp.maximum(al, al * NEG_SLOPE)
        svs[p][pl.ds(j, L)] = jnp.exp(al)

      @pl.loop(0, K, step=4)
      def _(r0):
        for u in range(4):
          r = r0 + u
          ridx = jnp.zeros((L,), jnp.int32) + r
          ssplat = plsc.load_gather(svs[p], [ridx])
          row = rowss[p].at[r]
          for c in range(d // L):
            sl = pl.ds(c * L, L)
            row[sl] = row[sl] * ssplat

    def start_scatter(p):
      pltpu.async_copy(rowss[p], acc_sh.at[sds[p]], semsc[p], add=True)
      pltpu.async_copy(svs[p], den_sh.at[sds[p]], semsc[p], add=True)

    def wait_scatter(p):
      pltpu.make_async_copy(rowss[p], acc_sh.at[sds[p]], semsc[p]).wait()
      pltpu.make_async_copy(svs[p], den_sh.at[sds[p]], semsc[p]).wait()

    start_idx(0, 0)
    start_idx(1, 1)
    wait_idx(0)
    start_gathers(0, 0)

    @pl.loop(0, (nb - 1) // 2)
    def _(g):
      for p in (0, 1):
        b = 2 * g + p
        q = 1 - p
        wait_gathers(p)
        wait_idx(q)

        @pl.when(b >= 1)
        def _():
          wait_scatter(q)

        start_gathers(b + 1, q)
        compute(p)

        @pl.when(b + 2 < nb)
        def _():
          start_idx(b + 2, p)

        start_scatter(p)

    wait_gathers(0)
    compute(0)
    start_scatter(0)
    wait_scatter(1)
    wait_scatter(0)

    plsc.subcore_barrier()

    stripe = pl.ds(sid * rows_per_tile, rows_per_tile)
    pltpu.sync_copy(acc_sh.at[stripe], out_hbm.at[cid].at[stripe])

    @pl.when(sid == 0)
    def _():
      pltpu.sync_copy(den_sh, den_hbm.at[cid])

  return sc_kernel(h, src, dst, a_edge, a_src, a_dst)


def _k2(out_p, h, a_src, a_dst, den_t, ae_sum, e_total):
  n, d = h.shape
  blk = 1000

  def body(p0_ref, p1_ref, h_ref, as_ref, ad_ref, dt_ref, sum_ref, o_ref):
    ae_mean = sum_ref[...] * (1.0 / e_total)
    v = as_ref[...] + ad_ref[...] + ae_mean
    v = jnp.maximum(v, v * NEG_SLOPE)
    s_self = jnp.exp(v)
    den = jnp.sum(dt_ref[...], axis=1, keepdims=True) + s_self
    numer = p0_ref[0] + p1_ref[0] + s_self * h_ref[...]
    o_ref[...] = numer / (den + 1e-16)

  return pl.pallas_call(
      body,
      grid=(n // blk,),
      in_specs=[
          pl.BlockSpec((1, blk, d), lambda i: (0, i, 0)),
          pl.BlockSpec((1, blk, d), lambda i: (1, i, 0)),
          pl.BlockSpec((blk, d), lambda i: (i, 0)),
          pl.BlockSpec((blk, 1), lambda i: (i, 0)),
          pl.BlockSpec((blk, 1), lambda i: (i, 0)),
          pl.BlockSpec((blk, NC), lambda i: (i, 0)),
          pl.BlockSpec((1, 1), lambda i: (0, 0)),
      ],
      out_specs=pl.BlockSpec((blk, d), lambda i: (i, 0)),
      out_shape=jax.ShapeDtypeStruct((n, d), jnp.float32),
  )(out_p, out_p, h, a_src, a_dst, den_t, ae_sum)


def kernel(x, edge_index, edge_attr, W, att_src, att_dst, W_edge, att_edge):
  e = edge_attr.shape[0]
  h, a_src2, a_dst2 = _k1a(x, W, att_src, att_dst)
  ae_row, ae_sum = _k1b(edge_attr, W_edge, att_edge)
  src = edge_index[0]
  dst = edge_index[1]
  n = h.shape[0]
  n_pad = 10240 if n == 10000 else ((n + 8 * NS - 1) // (8 * NS)) * 8 * NS
  out_p, den_p = _sc_edges(h, src, dst, ae_row.reshape(-1),
                           a_src2.reshape(-1), a_dst2.reshape(-1), n_pad)
  return _k2(out_p, h, a_src2, a_dst2, den_p.T, ae_sum, e)
